# baseline (speedup 1.0000x reference)
import functools

import jax
import jax.numpy as jnp
from jax import lax
from jax.experimental import pallas as pl
from jax.experimental.pallas import tpu as pltpu
from jax.experimental.pallas import tpu_sc as plsc

_N = 32 * 3 * 512 * 512
_NC, _NS, _L = 2, 16, 16
_NW = _NC * _NS
_HB = 16
_CHUNK_R = 128
_CHUNK = _CHUNK_R * 512
_NCHUNK = _HB * 3 * (512 // _CHUNK_R) // _NW


_sc_mesh = plsc.VectorSubcoreMesh(core_axis_name="c", subcore_axis_name="s")


@functools.partial(
    pl.kernel,
    out_type=jax.ShapeDtypeStruct((_NW, _L, 256), jnp.int32),
    mesh=_sc_mesh,
    compiler_params=pltpu.CompilerParams(
        needs_layout_passes=False, use_tc_tiling_on_sc=True
    ),
    scratch_types=[
        pltpu.VMEM((_CHUNK_R, 512), jnp.uint8),
        pltpu.VMEM((_CHUNK_R, 512), jnp.uint8),
        pltpu.VMEM((256 * _L,), jnp.int32),
        pltpu.VMEM((_L, 256), jnp.int32),
        pltpu.SemaphoreType.DMA,
        pltpu.SemaphoreType.DMA,
    ],
)
def _sc_hist(q_hbm, out_hbm, buf0, buf1, hist2, histt, sem0, sem1):
    wid = lax.axis_index("s") * _NC + lax.axis_index("c")

    zero = jnp.zeros((_L,), jnp.int32)

    @plsc.parallel_loop(0, 256 * _L, step=_L)
    def _zero_body(r):
        hist2[pl.ds(r, _L)] = zero

    bufs = (buf0, buf1)
    sems = (sem0, sem1)
    copies = [None, None]
    lane = lax.broadcasted_iota(jnp.int32, (_L,), 0)
    one = jnp.ones((_L,), jnp.int32)

    def _src(c):
        g = wid * _NCHUNK + c
        b = g // 12
        ch = (g % 12) // 4
        rb = g % 4
        return q_hbm.at[b, ch, pl.ds(rb * _CHUNK_R, _CHUNK_R), :]

    copies[0] = pltpu.async_copy(_src(0), buf0, sem0)
    for c in range(_NCHUNK):
        if c + 1 < _NCHUNK:
            nxt = (c + 1) % 2
            copies[nxt] = pltpu.async_copy(_src(c + 1), bufs[nxt], sems[nxt])
        copies[c % 2].wait()
        cur = bufs[c % 2]

        @plsc.parallel_loop(0, _CHUNK, step=4 * _L, unroll=4)
        def _chunk_body(i):
            x64 = cur[jnp.right_shift(i, 9), pl.ds(i & 511, 4 * _L)]
            v = plsc.bitcast(x64, jnp.int32)
            s0 = (jnp.left_shift(v, 4) & 0xFF0) | lane
            s1 = (jnp.right_shift(v, 4) & 0xFF0) | lane
            s2 = (jnp.right_shift(v, 12) & 0xFF0) | lane
            s3 = (jnp.right_shift(v, 20) & 0xFF0) | lane
            plsc.addupdate_scatter(hist2, [s0], one)
            plsc.addupdate_scatter(hist2, [s1], one)
            plsc.addupdate_scatter(hist2, [s2], one)
            plsc.addupdate_scatter(hist2, [s3], one)

    @plsc.parallel_loop(0, _L * 256, step=_L, unroll=4)
    def _tr_body(j):
        lane_out = jnp.right_shift(j, 8)
        bin_base = j & 255
        src = jnp.left_shift(bin_base + lane, 4) | lane_out
        histt[lane_out, pl.ds(bin_base, _L)] = plsc.load_gather(hist2, [src])

    pltpu.sync_copy(histt, out_hbm.at[wid])


def _tc_msq_body(o_ref, i_ref, q_ref, sq_ref, acc):
    step = pl.program_id(0)

    @pl.when(step == 0)
    def _init():
        acc[0, 0] = 0.0

    o = o_ref[...]
    d = o - i_ref[...]
    acc[0, 0] += jnp.sum(d * d)
    q_ref[...] = (o * 256.0).astype(jnp.uint8)

    @pl.when(step == _HB - 1)
    def _fini():
        sq_ref[0, 0] = acc[0, 0]


def _make_msq(half):
    return pl.pallas_call(
        _tc_msq_body,
        grid=(_HB,),
        in_specs=[
            pl.BlockSpec((1, 3, 512, 512), lambda g: (half * _HB + g, 0, 0, 0)),
            pl.BlockSpec((1, 3, 512, 512), lambda g: (half * _HB + g, 0, 0, 0)),
        ],
        out_specs=[
            pl.BlockSpec((1, 3, 512, 512), lambda g: (g, 0, 0, 0)),
            pl.BlockSpec(memory_space=pltpu.SMEM),
        ],
        out_shape=[
            jax.ShapeDtypeStruct((_HB, 3, 512, 512), jnp.uint8),
            jax.ShapeDtypeStruct((1, 1), jnp.float32),
        ],
        scratch_shapes=[pltpu.SMEM((1, 1), jnp.float32)],
    )


def _tc_combine_body(h0_ref, h1_ref, sq0_ref, sq1_ref, loss_ref, bpp_ref, dist_ref):
    counts = jnp.sum(h0_ref[...].astype(jnp.float32), axis=0) + jnp.sum(
        h1_ref[...].astype(jnp.float32), axis=0
    )
    total = jnp.sum(counts)
    p = counts / total
    p = jnp.clip(p, 1e-12, 1.0)
    ent = -jnp.sum(p * jnp.log2(p))
    bpp = ent / 32.0
    dist = (sq0_ref[0, 0] + sq1_ref[0, 0]) / float(_N)
    bpp_ref[0, 0] = bpp
    dist_ref[0, 0] = dist
    loss_ref[0, 0] = bpp + dist


@jax.jit
def kernel(outputs, inputs):
    q0, sq0 = _make_msq(0)(outputs, inputs)
    h0 = _sc_hist(q0)
    q1, sq1 = _make_msq(1)(outputs, inputs)
    h1 = _sc_hist(q1)
    loss, bpp, dist = pl.pallas_call(
        _tc_combine_body,
        in_specs=[
            pl.BlockSpec((_NW * _L, 256), lambda: (0, 0)),
            pl.BlockSpec((_NW * _L, 256), lambda: (0, 0)),
            pl.BlockSpec(memory_space=pltpu.SMEM),
            pl.BlockSpec(memory_space=pltpu.SMEM),
        ],
        out_specs=[
            pl.BlockSpec(memory_space=pltpu.SMEM),
            pl.BlockSpec(memory_space=pltpu.SMEM),
            pl.BlockSpec(memory_space=pltpu.SMEM),
        ],
        out_shape=[jax.ShapeDtypeStruct((1, 1), jnp.float32)] * 3,
    )(
        h0.reshape(_NW * _L, 256),
        h1.reshape(_NW * _L, 256),
        sq0,
        sq1,
    )
    return loss[0, 0], bpp[0, 0], dist[0, 0]
